# Initial kernel scaffold; baseline (speedup 1.0000x reference)
import jax
import jax.numpy as jnp
from jax.experimental import pallas as pl


def kernel(x, indices, down_row, down_col, down_val, W, b):
    raise NotImplementedError("write your pallas kernel here")


# speedup vs baseline: 12.1985x; 12.1985x over previous
import functools

import jax
import jax.numpy as jnp
from jax import lax
from jax.experimental import pallas as pl
from jax.experimental.pallas import tpu as pltpu
from jax.experimental.pallas import tpu_sc as plsc

_BS = 4
_N = 10000
_SEQ = 9
_INC = 128
_OUTC = 32
_NDOWN = 5000
_NNZ = 20000

_NC = 2
_NS = 16
_NW = _NC * _NS

_CH = 80
_CPW = 4
_NP = _NW * _CPW * _CH
_NCHUNK = _NP // _CH

_KCH = 128
_KPT = 10
_NKP = _NS * _KPT * _KCH
_NDP = 5120


def _mm_body(x_ref, v_ref, o_ref):
    xv = x_ref[...]
    for s in range(_SEQ):
        o_ref[s] = jnp.dot(xv, v_ref[s], preferred_element_type=jnp.float32)


def _matmul(x2, v):
    bn = 2000
    return pl.pallas_call(
        _mm_body,
        grid=(_BS * _N // bn,),
        in_specs=[
            pl.BlockSpec((bn, _INC), lambda i: (i, 0)),
            pl.BlockSpec((_SEQ, _INC, _OUTC), lambda i: (0, 0, 0)),
        ],
        out_specs=pl.BlockSpec((_SEQ, bn, _OUTC), lambda i: (0, i, 0)),
        out_shape=jax.ShapeDtypeStruct((_SEQ, _BS * _N, _OUTC), jnp.float32),
    )(x2, v)


def _gather_elu(ys2, idxc, bias):
    mesh = plsc.VectorSubcoreMesh(core_axis_name="c", subcore_axis_name="s")

    @functools.partial(
        pl.kernel,
        out_type=jax.ShapeDtypeStruct((_BS, _NP, _OUTC), jnp.float32),
        mesh=mesh,
        compiler_params=pltpu.CompilerParams(use_tc_tiling_on_sc=False),
        scratch_types=[
            pltpu.VMEM((_SEQ, _CH, _OUTC), jnp.float32),
            pltpu.VMEM((_CH, _OUTC), jnp.float32),
            pltpu.VMEM((_SEQ, _CH), jnp.int32),
            pltpu.VMEM((_OUTC,), jnp.float32),
            pltpu.SemaphoreType.DMA,
        ],
    )
    def k(ys_hbm, idx_hbm, b_hbm, h_hbm, gbuf, hbuf, idxv, biasv, sem):
        cid = lax.axis_index("c")
        sid = lax.axis_index("s")
        wid = sid * _NC + cid
        pltpu.sync_copy(b_hbm, biasv)
        b_lo = biasv[pl.ds(0, 16)]
        b_hi = biasv[pl.ds(16, 16)]
        for b in range(_BS):
            for j in range(_CPW):
                ch = wid * _CPW + j
                pltpu.sync_copy(idx_hbm.at[b, ch], idxv)
                descs = []
                for s in range(_SEQ):
                    descs.append(
                        pltpu.async_copy(ys_hbm.at[idxv.at[s]], gbuf.at[s], sem)
                    )
                for d in descs:
                    d.wait()

                @pl.loop(0, _CH)
                def _(n):
                    lo = gbuf[0, n, pl.ds(0, 16)]
                    hi = gbuf[0, n, pl.ds(16, 16)]
                    for s in range(1, _SEQ):
                        lo = lo + gbuf[s, n, pl.ds(0, 16)]
                        hi = hi + gbuf[s, n, pl.ds(16, 16)]
                    lo = lo + b_lo
                    hi = hi + b_hi
                    lo = jnp.where(lo > 0.0, lo, jnp.exp(lo) - 1.0)
                    hi = jnp.where(hi > 0.0, hi, jnp.exp(hi) - 1.0)
                    hbuf[n, pl.ds(0, 16)] = lo
                    hbuf[n, pl.ds(16, 16)] = hi

                pltpu.sync_copy(hbuf, h_hbm.at[b, pl.ds(ch * _CH, _CH)])

    return k(ys2, idxc, bias)


def _pool(h2, col_abs, row_abs, valx):
    mesh = plsc.VectorSubcoreMesh(core_axis_name="c", subcore_axis_name="s")

    @functools.partial(
        pl.kernel,
        out_type=jax.ShapeDtypeStruct((_BS, _NDP, _OUTC), jnp.float32),
        mesh=mesh,
        compiler_params=pltpu.CompilerParams(use_tc_tiling_on_sc=False),
        scratch_types=[
            pltpu.VMEM_SHARED((2 * _NDP, _OUTC), jnp.float32),
            pltpu.VMEM((_KCH, _OUTC), jnp.float32),
            pltpu.VMEM((_KPT, _KCH), jnp.int32),
            pltpu.VMEM((2, _KPT, _KCH), jnp.int32),
            pltpu.VMEM((_KPT, _KCH, _OUTC), jnp.float32),
            pltpu.SemaphoreType.DMA,
        ],
    )
    def k(h_hbm, col_hbm, row_hbm, val_hbm, out_hbm,
          accum, gbuf, colv, rowv, valv, sem):
        cid = lax.axis_index("c")
        sid = lax.axis_index("s")

        zero = jnp.zeros((16,), jnp.float32)

        @pl.loop(0, _KCH)
        def _(n):
            gbuf[n, pl.ds(0, 16)] = zero
            gbuf[n, pl.ds(16, 16)] = zero

        nzc = 2 * _NDP // _KCH // _NS
        for i in range(nzc):
            pltpu.sync_copy(
                gbuf, accum.at[pl.ds((sid * nzc + i) * _KCH, _KCH)]
            )
        pltpu.sync_copy(row_hbm.at[:, pl.ds(sid * _KPT, _KPT)], rowv)
        pltpu.sync_copy(val_hbm.at[pl.ds(sid * _KPT, _KPT)], valv)
        plsc.subcore_barrier()

        for bl in range(2):
            b = cid * 2 + bl
            pltpu.sync_copy(col_hbm.at[b, pl.ds(sid * _KPT, _KPT)], colv)
            for j in range(_KPT):
                pltpu.async_copy(h_hbm.at[colv.at[j]], gbuf, sem).wait()

                @pl.loop(0, _KCH)
                def _(n):
                    gbuf[n, pl.ds(0, 16)] = (
                        gbuf[n, pl.ds(0, 16)] * valv[j, n, pl.ds(0, 16)]
                    )
                    gbuf[n, pl.ds(16, 16)] = (
                        gbuf[n, pl.ds(16, 16)] * valv[j, n, pl.ds(16, 16)]
                    )

                pltpu.sync_copy(gbuf, accum.at[rowv.at[bl, j]], add=True)

        plsc.subcore_barrier()
        rpt = _NDP // _NS
        for bl in range(2):
            pltpu.sync_copy(
                accum.at[pl.ds(bl * _NDP + sid * rpt, rpt)],
                out_hbm.at[cid * 2 + bl, pl.ds(sid * rpt, rpt)],
            )

    return k(h2, col_abs, row_abs, valx)


def kernel(x, indices, down_row, down_col, down_val, W, b):
    x2 = x.reshape(_BS * _N, _INC)
    v = W.reshape(_OUTC, _SEQ, _INC).transpose(1, 2, 0)
    ys = _matmul(x2, v)
    ys2 = ys.reshape(_SEQ * _BS * _N, _OUTC)

    idxt = indices.astype(jnp.int32).T
    idxt = jnp.pad(idxt, ((0, 0), (0, _NP - _N)))
    offs = (jnp.arange(_SEQ, dtype=jnp.int32) * (_BS * _N))[None, :, None] + (
        jnp.arange(_BS, dtype=jnp.int32) * _N
    )[:, None, None]
    idxa = idxt[None, :, :] + offs
    idxc = idxa.reshape(_BS, _SEQ, _NCHUNK, _CH).transpose(0, 2, 1, 3)

    h = _gather_elu(ys2, idxc, b)
    h2 = h.reshape(_BS * _NP, _OUTC)

    colp = jnp.pad(down_col.astype(jnp.int32), (0, _NKP - _NNZ))
    rowp = jnp.pad(down_row.astype(jnp.int32), (0, _NKP - _NNZ))
    valp = jnp.pad(down_val, (0, _NKP - _NNZ))
    col_abs = (
        colp[None, :] + (jnp.arange(_BS, dtype=jnp.int32) * _NP)[:, None]
    ).reshape(_BS, _NS * _KPT, _KCH)
    row_abs = (
        rowp[None, :] + (jnp.arange(2, dtype=jnp.int32) * _NDP)[:, None]
    ).reshape(2, _NS * _KPT, _KCH)
    valx = jnp.broadcast_to(valp[:, None], (_NKP, _OUTC)).reshape(
        _NS * _KPT, _KCH, _OUTC
    )

    outp = _pool(h2, col_abs, row_abs, valx)
    return outp[:, :_NDOWN, :]


# speedup vs baseline: 22.5405x; 1.8478x over previous
import functools

import jax
import jax.numpy as jnp
from jax import lax
from jax.experimental import pallas as pl
from jax.experimental.pallas import tpu as pltpu
from jax.experimental.pallas import tpu_sc as plsc

_BS = 4
_N = 10000
_SEQ = 9
_INC = 128
_OUTC = 32
_LANES = _BS * _OUTC
_NDOWN = 5000
_NNZ = 20000

_NC = 2
_NS = 16
_NW = _NC * _NS

_CH = 40
_CPW = 8
_NP = _NW * _CPW * _CH
_NCHUNK = _NP // _CH

_KCH = 128
_KPT = 5
_NKCH = _NC * _NS * _KPT
_NKP = _NKCH * _KCH
_NDP = 5120


def _mm_body(x_ref, v_ref, o_ref):
    for s in range(_SEQ):
        cols = [
            jnp.dot(x_ref[bb], v_ref[s], preferred_element_type=jnp.float32)
            for bb in range(_BS)
        ]
        o_ref[s] = jnp.concatenate(cols, axis=-1)


def _matmul(x, v):
    bn = 2000
    return pl.pallas_call(
        _mm_body,
        grid=(_N // bn,),
        in_specs=[
            pl.BlockSpec((_BS, bn, _INC), lambda i: (0, i, 0)),
            pl.BlockSpec((_SEQ, _INC, _OUTC), lambda i: (0, 0, 0)),
        ],
        out_specs=pl.BlockSpec((_SEQ, bn, _LANES), lambda i: (0, i, 0)),
        out_shape=jax.ShapeDtypeStruct((_SEQ, _N, _LANES), jnp.float32),
    )(x, v)


def _gather_elu(ys3, idxf, bias4):
    mesh = plsc.VectorSubcoreMesh(core_axis_name="c", subcore_axis_name="s")

    @functools.partial(
        pl.kernel,
        out_type=jax.ShapeDtypeStruct((_NP, _LANES), jnp.float32),
        mesh=mesh,
        compiler_params=pltpu.CompilerParams(use_tc_tiling_on_sc=False),
        scratch_types=[
            pltpu.VMEM((2, _SEQ, _CH, _LANES), jnp.float32),
            pltpu.VMEM((2, _CH, _LANES), jnp.float32),
            pltpu.VMEM((_CPW * _SEQ * _CH,), jnp.int32),
            pltpu.VMEM((_LANES,), jnp.float32),
            pltpu.SemaphoreType.DMA,
            pltpu.SemaphoreType.DMA,
            pltpu.SemaphoreType.DMA,
            pltpu.SemaphoreType.DMA,
        ],
    )
    def k(ys_hbm, idx_hbm, b_hbm, h_hbm, gbuf, hbuf, idxv, biasv,
          sg0, sg1, sh0, sh1):
        cid = lax.axis_index("c")
        sid = lax.axis_index("s")
        wid = sid * _NC + cid
        gsems = [sg0, sg1]
        hsems = [sh0, sh1]
        pltpu.sync_copy(b_hbm, biasv)
        pltpu.sync_copy(
            idx_hbm.at[pl.ds(wid * _CPW * _SEQ * _CH, _CPW * _SEQ * _CH)],
            idxv,
        )
        bvs = [biasv[pl.ds(16 * i, 16)] for i in range(_LANES // 16)]

        def fire(j):
            p = j % 2
            descs = []
            for s in range(_SEQ):
                descs.append(
                    pltpu.async_copy(
                        ys_hbm.at[
                            idxv.at[pl.ds((j * _SEQ + s) * _CH, _CH)]
                        ],
                        gbuf.at[p, s],
                        gsems[p],
                    )
                )
            return descs

        gdescs = {0: fire(0)}
        hdescs = {}
        for j in range(_CPW):
            p = j % 2
            ch = wid * _CPW + j
            for d in gdescs.pop(j):
                d.wait()
            if j + 1 < _CPW:
                gdescs[j + 1] = fire(j + 1)
            if j - 2 in hdescs:
                hdescs.pop(j - 2).wait()

            @pl.loop(0, _CH)
            def _(n):
                for i in range(_LANES // 16):
                    acc = gbuf[p, 0, n, pl.ds(16 * i, 16)]
                    for s in range(1, _SEQ):
                        acc = acc + gbuf[p, s, n, pl.ds(16 * i, 16)]
                    acc = acc + bvs[i]
                    acc = jnp.where(acc > 0.0, acc, jnp.exp(acc) - 1.0)
                    hbuf[p, n, pl.ds(16 * i, 16)] = acc

            hdescs[j] = pltpu.async_copy(
                hbuf.at[p], h_hbm.at[pl.ds(ch * _CH, _CH)], hsems[p]
            )
        for j, d in hdescs.items():
            d.wait()

    return k(ys3, idxf, bias4)


def _pool(h, colf, rowc, valx):
    mesh = plsc.VectorSubcoreMesh(core_axis_name="c", subcore_axis_name="s")

    @functools.partial(
        pl.kernel,
        out_type=jax.ShapeDtypeStruct((_NC, _NDP, _LANES), jnp.float32),
        mesh=mesh,
        compiler_params=pltpu.CompilerParams(use_tc_tiling_on_sc=False),
        scratch_types=[
            pltpu.VMEM_SHARED((_NDP, _LANES), jnp.float32),
            pltpu.VMEM((2, _KCH, _LANES), jnp.float32),
            pltpu.VMEM((_KPT * _KCH,), jnp.int32),
            pltpu.VMEM((_KPT, _KCH), jnp.int32),
            pltpu.VMEM((_KPT, _KCH), jnp.float32),
            pltpu.SemaphoreType.DMA,
            pltpu.SemaphoreType.DMA,
            pltpu.SemaphoreType.DMA,
        ],
    )
    def k(h_hbm, col_hbm, row_hbm, val_hbm, part_hbm,
          accum, gbuf, colv, rowv, valv, sg0, sg1, ssem):
        cid = lax.axis_index("c")
        sid = lax.axis_index("s")
        cbase = cid * (_NS * _KPT) + sid * _KPT
        gsems = [sg0, sg1]

        zero = jnp.zeros((16,), jnp.float32)

        @pl.loop(0, _KCH)
        def _(n):
            for i in range(_LANES // 16):
                gbuf[0, n, pl.ds(16 * i, 16)] = zero

        rpt = _NDP // _NS
        pltpu.sync_copy(gbuf.at[0], accum.at[pl.ds(sid * rpt, _KCH)])
        pltpu.sync_copy(gbuf.at[0], accum.at[pl.ds(sid * rpt + _KCH, _KCH)])
        pltpu.sync_copy(
            gbuf.at[0, pl.ds(0, rpt - 2 * _KCH)],
            accum.at[pl.ds(sid * rpt + 2 * _KCH, rpt - 2 * _KCH)],
        )
        pltpu.sync_copy(col_hbm.at[pl.ds(cbase * _KCH, _KPT * _KCH)], colv)
        pltpu.sync_copy(row_hbm.at[pl.ds(cbase, _KPT)], rowv)
        pltpu.sync_copy(val_hbm.at[pl.ds(cbase, _KPT)], valv)
        plsc.subcore_barrier()

        def fire(j):
            p = j % 2
            return pltpu.async_copy(
                h_hbm.at[colv.at[pl.ds(j * _KCH, _KCH)]],
                gbuf.at[p],
                gsems[p],
            )

        gd = {0: fire(0)}
        scd = {}
        for j in range(_KPT):
            p = j % 2
            gd.pop(j).wait()
            if j - 1 in scd:
                scd.pop(j - 1).wait()
            if j + 1 < _KPT:
                gd[j + 1] = fire(j + 1)

            @pl.loop(0, _KCH // 16)
            def _(g):
                vv = valv[j, pl.ds(g * 16, 16)]

                @pl.loop(0, 16)
                def _(t):
                    n = g * 16 + t
                    vs = lax.gather(
                        vv,
                        jnp.full((16, 1), t, jnp.int32),
                        lax.GatherDimensionNumbers(
                            offset_dims=(),
                            collapsed_slice_dims=(0,),
                            start_index_map=(0,),
                        ),
                        (1,),
                        mode=lax.GatherScatterMode.PROMISE_IN_BOUNDS,
                    )
                    for i in range(_LANES // 16):
                        gbuf[p, n, pl.ds(16 * i, 16)] = (
                            gbuf[p, n, pl.ds(16 * i, 16)] * vs
                        )

            scd[j] = pltpu.async_copy(
                gbuf.at[p], accum.at[rowv.at[j]], ssem, add=True
            )
        scd.pop(_KPT - 1).wait()

        plsc.subcore_barrier()
        pltpu.sync_copy(
            accum.at[pl.ds(sid * rpt, rpt)],
            part_hbm.at[cid, pl.ds(sid * rpt, rpt)],
        )

    return k(h, colf, rowc, valx)


def _comb_body(p_ref, o_ref):
    s = p_ref[0] + p_ref[1]
    for b in range(_BS):
        o_ref[b] = s[:, b * _OUTC:(b + 1) * _OUTC]


def _combine(part):
    bn = 1280
    return pl.pallas_call(
        _comb_body,
        grid=(_NDP // bn,),
        in_specs=[pl.BlockSpec((_NC, bn, _LANES), lambda i: (0, i, 0))],
        out_specs=pl.BlockSpec((_BS, bn, _OUTC), lambda i: (0, i, 0)),
        out_shape=jax.ShapeDtypeStruct((_BS, _NDP, _OUTC), jnp.float32),
    )(part)


def kernel(x, indices, down_row, down_col, down_val, W, b):
    v = W.reshape(_OUTC, _SEQ, _INC).transpose(1, 2, 0)
    ys3 = _matmul(x, v).reshape(_SEQ * _N, _LANES)

    idxt = indices.astype(jnp.int32).T
    idxt = jnp.pad(idxt, ((0, 0), (0, _NP - _N)))
    idxa = idxt + (jnp.arange(_SEQ, dtype=jnp.int32) * _N)[:, None]
    idxf = idxa.reshape(_SEQ, _NCHUNK, _CH).transpose(1, 0, 2).reshape(-1)

    bias4 = jnp.tile(b, _BS)
    h = _gather_elu(ys3, idxf, bias4)

    colf = jnp.pad(down_col.astype(jnp.int32), (0, _NKP - _NNZ))
    rowc = jnp.pad(down_row.astype(jnp.int32), (0, _NKP - _NNZ)).reshape(
        _NKCH, _KCH
    )
    valx = jnp.pad(down_val, (0, _NKP - _NNZ)).reshape(_NKCH, _KCH)

    part = _pool(h, colf, rowc, valx)
    outp = _combine(part)
    return outp[:, :_NDOWN, :]
